# baseline (speedup 1.0000x reference)
import jax
import jax.numpy as jnp
from jax import lax
from jax.experimental import pallas as pl
from jax.experimental.pallas import tpu as pltpu

K = 32
KK = K * K
XMIN, XMAX = -15.0, 15.0
YMIN, YMAX = -15.0, 15.0
EPS = 1e-06


def _splat_body(coord_ref, feat_ref, out_ref, w_ref, s_scr):
    j = pl.program_id(1)

    @pl.when(j == 0)
    def _build_and_weights():
        cx = coord_ref[0, 0, :]
        cy = coord_ref[0, 1, :]
        P = cx.shape[0]
        gx = (cx - XMIN) / max(XMAX - XMIN, 1e-06) * (K - 1)
        gy = (cy - YMIN) / max(YMAX - YMIN, 1e-06) * (K - 1)
        x0 = jnp.floor(gx)
        y0 = jnp.floor(gy)
        x1 = x0 + 1.0
        y1 = y0 + 1.0
        wx1 = gx - x0
        wy1 = gy - y0
        wx0 = 1.0 - wx1
        wy0 = 1.0 - wy1
        cells = lax.broadcasted_iota(jnp.int32, (P, KK), 1)
        S = jnp.zeros((P, KK), dtype=jnp.float32)
        for nx, ny, w in ((x0, y0, wx0 * wy0), (x1, y0, wx1 * wy0),
                          (x0, y1, wx0 * wy1), (x1, y1, wx1 * wy1)):
            valid = ((nx >= 0) & (nx < K) & (ny >= 0) & (ny < K))
            idx = (jnp.clip(ny, 0, K - 1) * K + jnp.clip(nx, 0, K - 1)).astype(jnp.int32)
            wv = jnp.where(valid, w, 0.0)
            S = S + jnp.where(idx[:, None] == cells, wv[:, None], 0.0)
        colsum = jnp.sum(S, axis=0)
        s_scr[...] = S * (1.0 / jnp.clip(colsum, EPS, None))[None, :]
        w_ref[0, 0, :] = colsum

    res = jnp.dot(feat_ref[0], s_scr[...],
                  preferred_element_type=jnp.float32,
                  precision=lax.Precision.HIGHEST)
    t_blk, d = out_ref.shape[1], out_ref.shape[2]
    for y in range(K):
        out_ref[0, :, :, y, :] = res[:, y * K:(y + 1) * K].reshape(t_blk, d, K)


def kernel(patch_features, coord_map):
    b, t, d, hp, wp = patch_features.shape
    P = hp * wp
    TB = 2
    feats = patch_features.reshape(b, t * d, P)
    coords = coord_map.reshape(b, P, 2).transpose(0, 2, 1)

    grid = (b, t // TB)
    world, weights = pl.pallas_call(
        _splat_body,
        grid=grid,
        in_specs=[
            pl.BlockSpec((1, 2, P), lambda i, j: (i, 0, 0)),
            pl.BlockSpec((1, TB * d, P), lambda i, j: (i, j, 0)),
        ],
        out_specs=[
            pl.BlockSpec((1, TB, d, K, K), lambda i, j: (i, j, 0, 0, 0)),
            pl.BlockSpec((1, 1, KK), lambda i, j: (i, 0, 0)),
        ],
        out_shape=[
            jax.ShapeDtypeStruct((b, t, d, K, K), jnp.float32),
            jax.ShapeDtypeStruct((b, 1, KK), jnp.float32),
        ],
        scratch_shapes=[pltpu.VMEM((P, KK), jnp.float32)],
    )(coords, feats)
    weights = jnp.broadcast_to(weights.reshape(b, 1, 1, K, K),
                               (b, t, 1, K, K))
    return (world, weights)


# speedup vs baseline: 2.0543x; 2.0543x over previous
import jax
import jax.numpy as jnp
from jax import lax
from jax.experimental import pallas as pl
from jax.experimental.pallas import tpu as pltpu

K = 32
KK = K * K
XMIN, XMAX = -15.0, 15.0
YMIN, YMAX = -15.0, 15.0
EPS = 1e-06


def _splat_body(coord_ref, feat_ref, out_ref, w_ref, s_scr):
    j = pl.program_id(1)

    @pl.when(j == 0)
    def _build_and_weights():
        cx = coord_ref[0, 0, :]
        cy = coord_ref[0, 1, :]
        P = cx.shape[0]
        gx = (cx - XMIN) / max(XMAX - XMIN, 1e-06) * (K - 1)
        gy = (cy - YMIN) / max(YMAX - YMIN, 1e-06) * (K - 1)
        x0 = jnp.floor(gx)
        y0 = jnp.floor(gy)
        x1 = x0 + 1.0
        y1 = y0 + 1.0
        wx1 = gx - x0
        wy1 = gy - y0
        wx0 = 1.0 - wx1
        wy0 = 1.0 - wy1
        cells = lax.broadcasted_iota(jnp.int32, (P, KK), 1)
        S = jnp.zeros((P, KK), dtype=jnp.float32)
        for nx, ny, w in ((x0, y0, wx0 * wy0), (x1, y0, wx1 * wy0),
                          (x0, y1, wx0 * wy1), (x1, y1, wx1 * wy1)):
            valid = ((nx >= 0) & (nx < K) & (ny >= 0) & (ny < K))
            idx = (jnp.clip(ny, 0, K - 1) * K + jnp.clip(nx, 0, K - 1)).astype(jnp.int32)
            wv = jnp.where(valid, w, 0.0)
            S = S + jnp.where(idx[:, None] == cells, wv[:, None], 0.0)
        colsum = jnp.sum(S, axis=0)
        s_scr[...] = S * (1.0 / jnp.clip(colsum, EPS, None))[None, :]
        w_ref[0, 0, :] = colsum

    out_ref[0] = jnp.dot(feat_ref[0], s_scr[...],
                         preferred_element_type=jnp.float32,
                         precision=lax.Precision.DEFAULT)


def kernel(patch_features, coord_map):
    b, t, d, hp, wp = patch_features.shape
    P = hp * wp
    TD = t * d
    TDB = 256
    feats = patch_features.reshape(b, TD, P)
    coords = coord_map.reshape(b, P, 2).transpose(0, 2, 1)

    grid = (b, TD // TDB)
    world, weights = pl.pallas_call(
        _splat_body,
        grid=grid,
        in_specs=[
            pl.BlockSpec((1, 2, P), lambda i, j: (i, 0, 0)),
            pl.BlockSpec((1, TDB, P), lambda i, j: (i, j, 0)),
        ],
        out_specs=[
            pl.BlockSpec((1, TDB, KK), lambda i, j: (i, j, 0)),
            pl.BlockSpec((1, 1, KK), lambda i, j: (i, 0, 0)),
        ],
        out_shape=[
            jax.ShapeDtypeStruct((b, TD, KK), jnp.float32),
            jax.ShapeDtypeStruct((b, 1, KK), jnp.float32),
        ],
        scratch_shapes=[pltpu.VMEM((P, KK), jnp.float32)],
    )(coords, feats)
    world = world.reshape(b, t, d, K, K)
    weights = jnp.broadcast_to(weights.reshape(b, 1, 1, K, K),
                               (b, t, 1, K, K))
    return (world, weights)


# speedup vs baseline: 2.0755x; 1.0103x over previous
import jax
import jax.numpy as jnp
from jax import lax
from jax.experimental import pallas as pl
from jax.experimental.pallas import tpu as pltpu

K = 32
KK = K * K
XMIN, XMAX = -15.0, 15.0
YMIN, YMAX = -15.0, 15.0
EPS = 1e-06


def _splat_body(coord_ref, feat_ref, out_ref, w_ref, s_scr):
    j = pl.program_id(1)

    @pl.when(j == 0)
    def _build_and_weights():
        cxy = coord_ref[0]
        P = cxy.shape[0]
        cx = cxy[:, 0:1]
        cy = cxy[:, 1:2]
        gx = (cx - XMIN) / max(XMAX - XMIN, 1e-06) * (K - 1)
        gy = (cy - YMIN) / max(YMAX - YMIN, 1e-06) * (K - 1)
        x0 = jnp.floor(gx)
        y0 = jnp.floor(gy)
        x1 = x0 + 1.0
        y1 = y0 + 1.0
        wx1 = gx - x0
        wy1 = gy - y0
        wx0 = 1.0 - wx1
        wy0 = 1.0 - wy1
        cells = lax.broadcasted_iota(jnp.int32, (P, KK), 1)
        S = jnp.zeros((P, KK), dtype=jnp.float32)
        for nx, ny, w in ((x0, y0, wx0 * wy0), (x1, y0, wx1 * wy0),
                          (x0, y1, wx0 * wy1), (x1, y1, wx1 * wy1)):
            valid = ((nx >= 0) & (nx < K) & (ny >= 0) & (ny < K))
            idx = (jnp.clip(ny, 0, K - 1) * K + jnp.clip(nx, 0, K - 1)).astype(jnp.int32)
            wv = jnp.where(valid, w, 0.0)
            S = S + jnp.where(idx == cells, wv, 0.0)
        colsum = jnp.sum(S, axis=0)
        s_scr[...] = S * (1.0 / jnp.clip(colsum, EPS, None))[None, :]
        w_ref[0, 0, :] = colsum

    out_ref[0] = jnp.dot(feat_ref[0], s_scr[...],
                         preferred_element_type=jnp.float32,
                         precision=lax.Precision.DEFAULT)


def kernel(patch_features, coord_map):
    b, t, d, hp, wp = patch_features.shape
    P = hp * wp
    TD = t * d
    TDB = 512
    feats = patch_features.reshape(b, TD, P)
    coords = coord_map.reshape(b, P, 2)

    grid = (b, TD // TDB)
    world, weights = pl.pallas_call(
        _splat_body,
        grid=grid,
        in_specs=[
            pl.BlockSpec((1, P, 2), lambda i, j: (i, 0, 0)),
            pl.BlockSpec((1, TDB, P), lambda i, j: (i, j, 0)),
        ],
        out_specs=[
            pl.BlockSpec((1, TDB, KK), lambda i, j: (i, j, 0)),
            pl.BlockSpec((1, 1, KK), lambda i, j: (i, 0, 0)),
        ],
        out_shape=[
            jax.ShapeDtypeStruct((b, TD, KK), jnp.float32),
            jax.ShapeDtypeStruct((b, 1, KK), jnp.float32),
        ],
        scratch_shapes=[pltpu.VMEM((P, KK), jnp.float32)],
    )(coords, feats)
    world = world.reshape(b, t, d, K, K)
    weights = jnp.broadcast_to(weights.reshape(b, 1, 1, K, K),
                               (b, t, 1, K, K))
    return (world, weights)
